# baseline (speedup 1.0000x reference)
import functools

import jax
import jax.numpy as jnp
from jax import lax
from jax.experimental import pallas as pl
from jax.experimental.pallas import tpu as pltpu
from jax.experimental.pallas import tpu_sc as plsc


def kernel(inputs, pos_table):
    N, T = inputs.shape
    H = pos_table.shape[1]
    scale = float(H) ** 0.5
    NC, NS = 2, 16
    NW = NC * NS
    rows_per_w = N // NW

    mesh = plsc.VectorSubcoreMesh(core_axis_name="c", subcore_axis_name="s")

    @functools.partial(
        pl.kernel,
        mesh=mesh,
        out_type=jax.ShapeDtypeStruct((N, T, H), jnp.float32),
        scratch_types=[
            pltpu.VMEM((T, H), jnp.float32),
            pltpu.SemaphoreType.DMA,
        ],
    )
    def k(tab_hbm, out_hbm, buf, sem):
        wid = lax.axis_index("s") * NC + lax.axis_index("c")
        pltpu.sync_copy(tab_hbm, buf)

        @pl.loop(0, T)
        def _(t):
            @pl.loop(0, H, step=16)
            def _(h):
                slc = (pl.ds(t, 1), pl.ds(h, 16))
                buf.at[*slc][...] = buf.at[*slc][...] * scale

        base = wid * rows_per_w

        @pl.loop(0, rows_per_w)
        def _(i):
            pltpu.async_copy(buf, out_hbm.at[base + i], sem)

        @pl.loop(0, rows_per_w)
        def _(i):
            pltpu.make_async_copy(buf, out_hbm.at[base + i], sem).wait()

    return k(pos_table)


# speedup vs baseline: 1.0723x; 1.0723x over previous
import functools

import jax
import jax.numpy as jnp
from jax import lax
from jax.experimental import pallas as pl
from jax.experimental.pallas import tpu as pltpu
from jax.experimental.pallas import tpu_sc as plsc


def kernel(inputs, pos_table):
    N, T = inputs.shape
    H = pos_table.shape[1]
    D = T * H
    scale = float(H) ** 0.5

    NC, NS = 2, 16
    NW = NC * NS
    K = 4
    CH = N // K
    rows_per_w = CH // NW

    mesh = plsc.VectorSubcoreMesh(core_axis_name="c", subcore_axis_name="s")

    def make_chunk_kernel(chunk_idx):
        @functools.partial(
            pl.kernel,
            mesh=mesh,
            out_type=jax.ShapeDtypeStruct((CH, D), jnp.float32),
            scratch_types=[
                pltpu.VMEM((D,), jnp.float32),
                pltpu.SemaphoreType.DMA,
            ],
        )
        def sc_chunk(tab_hbm, out_hbm, buf, sem):
            wid = lax.axis_index("s") * NC + lax.axis_index("c")
            pltpu.sync_copy(tab_hbm, buf)

            @pl.loop(0, D, step=16)
            def _(i):
                buf.at[pl.ds(i, 16)][...] = buf.at[pl.ds(i, 16)][...] * scale

            base = ((wid + chunk_idx) % NW) * rows_per_w

            @pl.loop(0, rows_per_w)
            def _(r):
                pltpu.async_copy(buf, out_hbm.at[base + r], sem)

            @pl.loop(0, rows_per_w)
            def _(r):
                pltpu.make_async_copy(buf, out_hbm.at[base + r], sem).wait()

        return sc_chunk

    tab_flat = pos_table.reshape(D)
    parts = []
    for i in range(K):
        parts.append(make_chunk_kernel(i)(tab_flat).reshape(CH, T, H))
    return jnp.concatenate(parts, axis=0)
